# Initial kernel scaffold; baseline (speedup 1.0000x reference)
import jax
import jax.numpy as jnp
from jax.experimental import pallas as pl


def kernel(x, edge_index, edge_attr, params):
    raise NotImplementedError("write your pallas kernel here")


# speedup vs baseline: 13.8087x; 13.8087x over previous
import functools

import jax
import jax.numpy as jnp
from jax import lax
from jax.experimental import pallas as pl
from jax.experimental.pallas import tpu as pltpu
from jax.experimental.pallas import tpu_sc as plsc

NODE = 5000
NTOT = 10000
E = 320000
D = 128
HD = 64
NSEG = 20000
NC, NS, L = 2, 16, 16
SRCW = 144
DSTW = 80
CH = 80
EPT_A = E // NS
EPC = E // NC
EPT_B = EPC // NS
NB1 = 20
BP = 256
NPAD = 5120
BC = 200
NBC = NODE // BC
NAMES = ("s2s", "t2t", "s2t", "t2s")


def _pre2_body(epw, epb, attb, a3, vt_ref, c_ref):
    for n in range(4):
        vt_ref[n:n + 1, :] = jnp.dot(epw[n:n + 1, :], a3[n],
                                     preferred_element_type=jnp.float32)
        c_ref[n:n + 1, :] = jnp.dot(epb[n:n + 1, :], a3[n],
                                    preferred_element_type=jnp.float32) + attb[n:n + 1, :]


def _pre2(epw, epb, attb, a3):
    return pl.pallas_call(
        _pre2_body,
        out_shape=(jax.ShapeDtypeStruct((4, D), jnp.float32),
                   jax.ShapeDtypeStruct((4, D), jnp.float32)),
    )(epw, epb, attb, a3)


def _pre1_body(x_ref, w_ref, b_ref, wc_ref, bc_ref, h_ref, p_ref):
    h = jnp.dot(x_ref[0], w_ref[0], preferred_element_type=jnp.float32) + b_ref[0]
    h_ref[0] = h
    p_ref[0] = jnp.dot(h, wc_ref[0], preferred_element_type=jnp.float32) + bc_ref[0]


def _pre1(x2, w2, b2, wcat, bcat):
    return pl.pallas_call(
        _pre1_body,
        grid=(2, NB1),
        in_specs=[
            pl.BlockSpec((1, BP, D), lambda t, i: (t, i, 0)),
            pl.BlockSpec((1, D, D), lambda t, i: (t, 0, 0)),
            pl.BlockSpec((1, 1, D), lambda t, i: (t, 0, 0)),
            pl.BlockSpec((1, D, 640), lambda t, i: (t, 0, 0)),
            pl.BlockSpec((1, 1, 640), lambda t, i: (t, 0, 0)),
        ],
        out_specs=(
            pl.BlockSpec((1, BP, D), lambda t, i: (t, i, 0)),
            pl.BlockSpec((1, BP, 640), lambda t, i: (t, i, 0)),
        ),
        out_shape=(jax.ShapeDtypeStruct((2, NPAD, D), jnp.float32),
                   jax.ShapeDtypeStruct((2, NPAD, 640), jnp.float32)),
    )(x2, w2, b2, wcat, bcat)


def _sc_body(u_hbm, v_hbm, ea_hbm, src0_hbm, src1_hbm, dst0_hbm, dst1_hbm,
             asrc_hbm, adst_hbm, vt_hbm, z1_hbm, z2_hbm, out_hbm, sout_hbm,
             vt_v,
             uc_v, vc_v, eac_v, usel_v, vsel_v, exb_v, wbuf_v, qoff_v,
             sgi_v, asb_v, adb_v, sgb_v,
             srcrows_v, dstrows_v, msg_v,
             s_sp, agg_sp, sem):
    cid = lax.axis_index("c")
    sid = lax.axis_index("s")
    iot = lax.iota(jnp.int32, L)
    ONE = jnp.int32(1)
    ZERO = jnp.int32(0)

    pltpu.sync_copy(vt_hbm, vt_v)

    @pl.when(sid == 0)
    def _():
        pltpu.sync_copy(z1_hbm, s_sp)
    plsc.subcore_barrier()

    def chunk_a(i, _):
        base = pl.multiple_of(sid * EPT_A + i * CH, CH)
        pltpu.sync_copy(u_hbm.at[pl.ds(base, CH)], uc_v)
        pltpu.sync_copy(v_hbm.at[pl.ds(base, CH)], vc_v)
        for g in range(CH // L):
            ul = uc_v[pl.ds(g * L, L)]
            vl = vc_v[pl.ds(g * L, L)]
            su = jnp.where(ul >= NODE, ONE, ZERO)
            dv = jnp.where(vl >= NODE, ONE, ZERO)
            usel_v[pl.ds(g * L, L)] = ul + NTOT * dv
            vsel_v[pl.ds(g * L, L)] = vl + NTOT * su
        pltpu.async_copy(asrc_hbm.at[usel_v], asb_v, sem).wait()
        pltpu.async_copy(adst_hbm.at[vsel_v], adb_v, sem).wait()
        for g in range(CH // L):
            a = asb_v[pl.ds(g * L, L)] + adb_v[pl.ds(g * L, L)]
            a = jnp.where(a < 0.0, 0.2 * a, a)
            exb_v[pl.ds(g * L, L)] = jnp.exp(a)
        pltpu.sync_copy(exb_v, s_sp.at[vsel_v], add=True)
        return 0

    lax.fori_loop(0, EPT_A // CH, chunk_a, 0)
    plsc.subcore_barrier()

    @pl.when(sid == 0)
    def _():
        soff = pl.multiple_of(cid * NSEG, 8)
        pltpu.sync_copy(s_sp, sout_hbm.at[pl.ds(soff, NSEG)])
    plsc.subcore_barrier()

    for half in range(2):
        src_h = (src0_hbm, src1_hbm)[half]
        dst_h = (dst0_hbm, dst1_hbm)[half]
        @pl.when(sid < NS - 1)
        def _():
            off = pl.multiple_of(sid * 1248, 8)
            pltpu.sync_copy(z2_hbm.at[pl.ds(off, 1248)],
                            agg_sp.at[pl.ds(off, 1248)])

        @pl.when(sid == NS - 1)
        def _():
            pltpu.sync_copy(z2_hbm.at[pl.ds((NS - 1) * 1248, NSEG - (NS - 1) * 1248)],
                            agg_sp.at[pl.ds((NS - 1) * 1248, NSEG - (NS - 1) * 1248)])
        plsc.subcore_barrier()

        def chunk_b(i, _):
            base = pl.multiple_of(cid * EPC + sid * EPT_B + i * CH, CH)
            pltpu.sync_copy(u_hbm.at[pl.ds(base, CH)], uc_v)
            pltpu.sync_copy(v_hbm.at[pl.ds(base, CH)], vc_v)
            pltpu.sync_copy(ea_hbm.at[pl.ds(base, CH)], eac_v)
            for g in range(CH // L):
                ul = uc_v[pl.ds(g * L, L)]
                vl = vc_v[pl.ds(g * L, L)]
                su = jnp.where(ul >= NODE, ONE, ZERO)
                dv = jnp.where(vl >= NODE, ONE, ZERO)
                usel = ul + NTOT * dv
                vsel = vl + NTOT * su
                usel_v[pl.ds(g * L, L)] = usel
                vsel_v[pl.ds(g * L, L)] = vsel
                sgi_v[pl.ds(g * L, L)] = cid * NSEG + vsel
                qoff_v[pl.ds(g * L, L)] = (3 * su + 2 * dv - 4 * su * dv) * D + half * HD
            pltpu.async_copy(src_h.at[usel_v], srcrows_v, sem).wait()
            pltpu.async_copy(dst_h.at[vsel_v], dstrows_v, sem).wait()
            pltpu.async_copy(sout_hbm.at[sgi_v], sgb_v, sem).wait()
            for g in range(CH // L):
                le = iot + g * L
                a = (plsc.load_gather(srcrows_v, [le, jnp.full((L,), HD * 2, jnp.int32)])
                     + plsc.load_gather(dstrows_v, [le, jnp.full((L,), HD, jnp.int32)]))
                a = jnp.where(a < 0.0, 0.2 * a, a)
                sg = sgb_v[pl.ds(g * L, L)]
                wbuf_v[pl.ds(g * L, L)] = jnp.exp(a) / (sg + 1e-16)

            def edge_e(e, _):
                ef = jnp.full((L,), e, jnp.int32)
                we = plsc.load_gather(wbuf_v, [ef])
                eae = plsc.load_gather(eac_v, [ef])
                qov = plsc.load_gather(qoff_v, [ef])
                for j in range(HD // L):
                    col = iot + j * L
                    hv = plsc.load_gather(srcrows_v, [ef, col])
                    p1 = plsc.load_gather(srcrows_v, [ef, col + HD])
                    p2 = plsc.load_gather(dstrows_v, [ef, col])
                    vtv = plsc.load_gather(vt_v, [qov + col])
                    plsc.store_scatter(msg_v, [ef, col],
                                       hv * (p1 + p2 + eae * vtv) * we)
                return 0

            lax.fori_loop(0, CH, edge_e, 0)
            pltpu.sync_copy(msg_v, agg_sp.at[vsel_v], add=True)
            return 0

        lax.fori_loop(0, EPT_B // CH, chunk_b, 0)
        plsc.subcore_barrier()
        @pl.when(sid < NS - 1)
        def _():
            off = pl.multiple_of(sid * 1248, 8)
            pltpu.sync_copy(agg_sp.at[pl.ds(off, 1248)],
                            out_hbm.at[cid, half, pl.ds(off, 1248)])

        @pl.when(sid == NS - 1)
        def _():
            pltpu.sync_copy(agg_sp.at[pl.ds((NS - 1) * 1248, NSEG - (NS - 1) * 1248)],
                            out_hbm.at[cid, half, pl.ds((NS - 1) * 1248, NSEG - (NS - 1) * 1248)])
        plsc.subcore_barrier()


def _sc_edges(u, v, ea, src0, src1, dst0, dst1, asrc, adst, vtf, z1, z2):
    mesh = plsc.VectorSubcoreMesh(core_axis_name="c", subcore_axis_name="s",
                                  num_cores=NC, num_subcores=NS)
    fn = functools.partial(
        pl.kernel,
        compiler_params=pltpu.CompilerParams(use_tc_tiling_on_sc=False,
                                             needs_layout_passes=False),
        out_type=(jax.ShapeDtypeStruct((NC, 2, NSEG, HD), jnp.float32),
                  jax.ShapeDtypeStruct((NC * NSEG,), jnp.float32)),
        mesh=mesh,
        scratch_types=[
            pltpu.VMEM((4 * D,), jnp.float32),
            pltpu.VMEM((CH,), jnp.int32),
            pltpu.VMEM((CH,), jnp.int32),
            pltpu.VMEM((CH,), jnp.float32),
            pltpu.VMEM((CH,), jnp.int32),
            pltpu.VMEM((CH,), jnp.int32),
            pltpu.VMEM((CH,), jnp.float32),
            pltpu.VMEM((CH,), jnp.float32),
            pltpu.VMEM((CH,), jnp.int32),
            pltpu.VMEM((CH,), jnp.int32),
            pltpu.VMEM((CH,), jnp.float32),
            pltpu.VMEM((CH,), jnp.float32),
            pltpu.VMEM((CH,), jnp.float32),
            pltpu.VMEM((CH, SRCW), jnp.float32),
            pltpu.VMEM((CH, DSTW), jnp.float32),
            pltpu.VMEM((CH, HD), jnp.float32),
            pltpu.VMEM_SHARED((NSEG,), jnp.float32),
            pltpu.VMEM_SHARED((NSEG, HD), jnp.float32),
            pltpu.SemaphoreType.DMA,
        ],
    )(_sc_body)
    aggp, _ = fn(u, v, ea, src0, src1, dst0, dst1, asrc, adst, vtf, z1, z2)
    return aggp


def _c1_body(av0, av1, h_ref, w1, w2, bb, kw, kb, qq, o_ref, ps_ref):
    hb = h_ref[...]
    for vv in range(2):
        avr = (av0, av1)[vv]
        aggr = jnp.concatenate(
            [avr[0, 0] + avr[1, 0], avr[0, 1] + avr[1, 1]], axis=-1)
        upd = (jnp.dot(aggr, w1[0, vv], preferred_element_type=jnp.float32)
               + jnp.dot(hb, w2[0, vv], preferred_element_type=jnp.float32)
               + bb[0, vv])
        o = jnp.tanh(jnp.maximum(upd, 0.0))
        o_ref[0, vv] = o
        kl = jnp.tanh(jnp.dot(o, kw[0], preferred_element_type=jnp.float32) + kb[0])
        ps_ref[0, 0, vv] = jnp.sum(qq[0] * kl, axis=0)


def _c1(aggp, hall, w1, w2, bb, kw, kb, qq):
    def off0(nt, i):
        return (0, 0, jnp.where(nt == 0, 0, 15000 // BC) + i, 0)

    def off1(nt, i):
        return (0, 0, jnp.where(nt == 0, 10000 // BC, 5000 // BC) + i, 0)

    return pl.pallas_call(
        _c1_body,
        grid=(2, NBC),
        in_specs=[
            pl.BlockSpec((2, 2, BC, HD), off0),
            pl.BlockSpec((2, 2, BC, HD), off1),
            pl.BlockSpec((BC, D), lambda nt, i: (nt * NBC + i, 0)),
            pl.BlockSpec((1, 2, D, D), lambda nt, i: (nt, 0, 0, 0)),
            pl.BlockSpec((1, 2, D, D), lambda nt, i: (nt, 0, 0, 0)),
            pl.BlockSpec((1, 2, 1, D), lambda nt, i: (nt, 0, 0, 0)),
            pl.BlockSpec((1, D, D), lambda nt, i: (nt, 0, 0)),
            pl.BlockSpec((1, 1, D), lambda nt, i: (nt, 0, 0)),
            pl.BlockSpec((1, 1, D), lambda nt, i: (nt, 0, 0)),
        ],
        out_specs=(
            pl.BlockSpec((1, 2, BC, D), lambda nt, i: (nt, 0, i, 0)),
            pl.BlockSpec((1, 1, 2, D), lambda nt, i: (nt, i, 0, 0)),
        ),
        out_shape=(jax.ShapeDtypeStruct((2, 2, NODE, D), jnp.float32),
                   jax.ShapeDtypeStruct((2, NBC, 2, D), jnp.float32)),
    )(aggp, aggp, hall, w1, w2, bb, kw, kb, qq)


def _c2_body(o_ref, ps_ref, out_ref):
    p0 = jnp.sum(ps_ref[0, :, 0, :]) * (1.0 / NODE)
    p1 = jnp.sum(ps_ref[0, :, 1, :]) * (1.0 / NODE)
    m = jnp.maximum(p0, p1)
    e0 = jnp.exp(p0 - m)
    e1 = jnp.exp(p1 - m)
    a0 = e0 / (e0 + e1)
    a1 = e1 / (e0 + e1)
    out_ref[...] = a0 * o_ref[0, 0] + a1 * o_ref[0, 1]


def _c2(o, ps):
    return pl.pallas_call(
        _c2_body,
        grid=(2, NBC),
        in_specs=[
            pl.BlockSpec((1, 2, BC, D), lambda nt, i: (nt, 0, i, 0)),
            pl.BlockSpec((1, NBC, 2, D), lambda nt, i: (nt, 0, 0, 0)),
        ],
        out_specs=pl.BlockSpec((BC, D), lambda nt, i: (nt * NBC + i, 0)),
        out_shape=jax.ShapeDtypeStruct((NTOT, D), jnp.float32),
    )(o, ps)


def kernel(x, edge_index, edge_attr, params):
    xb = x[0]
    u = edge_index[0, 0]
    v = edge_index[0, 1]
    eaf = edge_attr[0, :, 0]
    p = params

    epw = jnp.stack([p["ep_W_" + n][0] for n in NAMES])
    epb = jnp.stack([p["ep_b_" + n] for n in NAMES])
    attb = jnp.stack([p["att_b_" + n] for n in NAMES])
    a3 = jnp.stack([p["att_W_" + n][2 * D:] for n in NAMES])
    vt, cc = _pre2(epw, epb, attb, a3)

    zpad = jnp.zeros((NPAD - NODE, D), jnp.float32)
    x2 = jnp.stack([jnp.concatenate([xb[:NODE], zpad]),
                    jnp.concatenate([xb[NODE:], zpad])])
    w2 = jnp.stack([p["proj_W_s"], p["proj_W_t"]])
    b2 = jnp.stack([p["proj_b_s"], p["proj_b_t"]])[:, None, :]

    def wcat_for(nt):
        n2s = nt + "2s"
        n2t = nt + "2t"
        fs = "s2" + nt
        ft = "t2" + nt
        scal = jnp.zeros((D, D), jnp.float32)
        scal = scal.at[:, 0].set(p["lin_src_" + n2s])
        scal = scal.at[:, 1].set(p["lin_src_" + n2t])
        scal = scal.at[:, 2].set(p["lin_dst_" + fs])
        scal = scal.at[:, 3].set(p["lin_dst_" + ft])
        return jnp.concatenate([
            p["att_W_" + n2s][:D], p["att_W_" + n2t][:D],
            p["att_W_" + fs][D:2 * D], p["att_W_" + ft][D:2 * D], scal], axis=1)

    wcat = jnp.stack([wcat_for("s"), wcat_for("t")])
    ni = {n: i for i, n in enumerate(NAMES)}

    def bcat_for(nt):
        z = jnp.zeros((D,), jnp.float32)
        return jnp.concatenate([z, z, cc[ni["s2" + nt]], cc[ni["t2" + nt]], z])

    bcat = jnp.stack([bcat_for("s"), bcat_for("t")])[:, None, :]
    hout, pout = _pre1(x2, w2, b2, wcat, bcat)
    hs, ht = hout[0, :NODE], hout[1, :NODE]
    ps_, pt_ = pout[0, :NODE], pout[1, :NODE]
    hall = jnp.concatenate([hs, ht])

    z15 = jnp.zeros((NODE, SRCW - 2 * HD - 1), jnp.float32)

    def src_tab(h):
        c = slice(h * HD, h * HD + HD)
        blks = []
        for dtv in range(2):
            a_col = 512 + dtv
            p1c = slice(dtv * D + h * HD, dtv * D + h * HD + HD)
            for hh, pp in ((hs, ps_), (ht, pt_)):
                blks.append(jnp.concatenate(
                    [hh[:, c], pp[:, p1c], pp[:, a_col][:, None], z15], axis=1))
        return jnp.concatenate(blks)

    def dst_tab(h):
        blks = []
        for stv in range(2):
            a_col = 514 + stv
            p2c = slice(2 * D + stv * D + h * HD, 2 * D + stv * D + h * HD + HD)
            for pp in (ps_, pt_):
                blks.append(jnp.concatenate(
                    [pp[:, p2c], pp[:, a_col][:, None], z15], axis=1))
        return jnp.concatenate(blks)

    src0, src1 = src_tab(0), src_tab(1)
    dst0, dst1 = dst_tab(0), dst_tab(1)
    asrc = jnp.concatenate([ps_[:, 512], pt_[:, 512], ps_[:, 513], pt_[:, 513]])
    adst = jnp.concatenate([ps_[:, 514], pt_[:, 514], ps_[:, 515], pt_[:, 515]])
    vtf = vt.reshape(-1)
    z1 = jnp.zeros((NSEG,), jnp.float32)
    z2 = jnp.zeros((NSEG, HD), jnp.float32)

    aggp = _sc_edges(u, v, eaf, src0, src1, dst0, dst1, asrc, adst, vtf, z1, z2)

    vorder = {"s": ("s2s", "t2s"), "t": ("t2t", "s2t")}
    w1 = jnp.stack([jnp.stack([p["agg_W_" + n][:D] for n in vorder[nt]]) for nt in "st"])
    w2c = jnp.stack([jnp.stack([p["agg_W_" + n][D:] for n in vorder[nt]]) for nt in "st"])
    bb = jnp.stack([jnp.stack([p["agg_b_" + n][None, :] for n in vorder[nt]]) for nt in "st"])
    kw = jnp.stack([p["k_W_s"], p["k_W_t"]])
    kb = jnp.stack([p["k_b_s"], p["k_b_t"]])[:, None, :]
    qq = jnp.stack([p["q_s"], p["q_t"]])

    o, psum = _c1(aggp, hall, w1, w2c, bb, kw, kb, qq)
    out = _c2(o, psum)
    return out[None]


# speedup vs baseline: 21.1674x; 1.5329x over previous
import functools

import jax
import jax.numpy as jnp
from jax import lax
from jax.experimental import pallas as pl
from jax.experimental.pallas import tpu as pltpu
from jax.experimental.pallas import tpu_sc as plsc

NODE = 5000
NTOT = 10000
E = 320000
D = 128
HD = 64
NSEG = 20000
NC, NS, L = 2, 16, 16
SRCW = 144
DSTW = 80
CH = 80
EPT_A = E // NS
EPC = E // NC
EPT_B = EPC // NS
NB1 = 20
BP = 256
NPAD = 5120
BC = 200
NBC = NODE // BC
NAMES = ("s2s", "t2t", "s2t", "t2s")


def _pre2_body(epw, epb, attb, a3, vt_ref, c_ref):
    for n in range(4):
        vt_ref[n:n + 1, :] = jnp.dot(epw[n:n + 1, :], a3[n],
                                     preferred_element_type=jnp.float32)
        c_ref[n:n + 1, :] = jnp.dot(epb[n:n + 1, :], a3[n],
                                    preferred_element_type=jnp.float32) + attb[n:n + 1, :]


def _pre2(epw, epb, attb, a3):
    return pl.pallas_call(
        _pre2_body,
        out_shape=(jax.ShapeDtypeStruct((4, D), jnp.float32),
                   jax.ShapeDtypeStruct((4, D), jnp.float32)),
    )(epw, epb, attb, a3)


def _pre1_body(x_ref, w_ref, b_ref, wc_ref, bc_ref, h_ref, p_ref):
    h = jnp.dot(x_ref[0], w_ref[0], preferred_element_type=jnp.float32) + b_ref[0]
    h_ref[0] = h
    p_ref[0] = jnp.dot(h, wc_ref[0], preferred_element_type=jnp.float32) + bc_ref[0]


def _pre1(x2, w2, b2, wcat, bcat):
    return pl.pallas_call(
        _pre1_body,
        grid=(2, NB1),
        in_specs=[
            pl.BlockSpec((1, BP, D), lambda t, i: (t, i, 0)),
            pl.BlockSpec((1, D, D), lambda t, i: (t, 0, 0)),
            pl.BlockSpec((1, 1, D), lambda t, i: (t, 0, 0)),
            pl.BlockSpec((1, D, 640), lambda t, i: (t, 0, 0)),
            pl.BlockSpec((1, 1, 640), lambda t, i: (t, 0, 0)),
        ],
        out_specs=(
            pl.BlockSpec((1, BP, D), lambda t, i: (t, i, 0)),
            pl.BlockSpec((1, BP, 640), lambda t, i: (t, i, 0)),
        ),
        out_shape=(jax.ShapeDtypeStruct((2, NPAD, D), jnp.float32),
                   jax.ShapeDtypeStruct((2, NPAD, 640), jnp.float32)),
    )(x2, w2, b2, wcat, bcat)


NCHA = EPT_A // CH
NCHB = EPT_B // CH


def _sc_body(u_hbm, v_hbm, ea_hbm, src0_hbm, src1_hbm, dst0_hbm, dst1_hbm,
             asrc_hbm, adst_hbm, vt_hbm, z1_hbm, z2_hbm, out_hbm, sout_hbm,
             vt_v, uc_v, vc_v, exb_v, wbuf_v, msg_v,
             eac0_v, eac1_v, usel0_v, usel1_v, vsel0_v, vsel1_v,
             sgi0_v, sgi1_v, qoff0_v, qoff1_v, asb0_v, asb1_v,
             adb0_v, adb1_v, sgb0_v, sgb1_v,
             srcrows0_v, srcrows1_v, dstrows0_v, dstrows1_v,
             s_sp, agg_sp, sem):
    cid = lax.axis_index("c")
    sid = lax.axis_index("s")
    iot = lax.iota(jnp.int32, L)
    ONE = jnp.int32(1)
    ZERO = jnp.int32(0)
    eac2 = (eac0_v, eac1_v)
    usel2 = (usel0_v, usel1_v)
    vsel2 = (vsel0_v, vsel1_v)
    sgi2 = (sgi0_v, sgi1_v)
    qoff2 = (qoff0_v, qoff1_v)
    asb2 = (asb0_v, asb1_v)
    adb2 = (adb0_v, adb1_v)
    sgb2 = (sgb0_v, sgb1_v)
    srcrows2 = (srcrows0_v, srcrows1_v)
    dstrows2 = (dstrows0_v, dstrows1_v)

    pltpu.sync_copy(vt_hbm, vt_v)

    @pl.when(sid == 0)
    def _():
        pltpu.sync_copy(z1_hbm, s_sp)
    plsc.subcore_barrier()

    def fire_a(i, p):
        base = pl.multiple_of(sid * EPT_A + i * CH, CH)
        pltpu.sync_copy(u_hbm.at[pl.ds(base, CH)], uc_v)
        pltpu.sync_copy(v_hbm.at[pl.ds(base, CH)], vc_v)
        for g in range(CH // L):
            ul = uc_v[pl.ds(g * L, L)]
            vl = vc_v[pl.ds(g * L, L)]
            su = jnp.where(ul >= NODE, ONE, ZERO)
            dv = jnp.where(vl >= NODE, ONE, ZERO)
            usel2[p][pl.ds(g * L, L)] = ul + NTOT * dv
            vsel2[p][pl.ds(g * L, L)] = vl + NTOT * su
        pltpu.async_copy(asrc_hbm.at[usel2[p]], asb2[p], sem)
        pltpu.async_copy(adst_hbm.at[vsel2[p]], adb2[p], sem)

    def compute_a(i, p):
        pltpu.make_async_copy(asrc_hbm.at[usel2[p]], asb2[p], sem).wait()
        pltpu.make_async_copy(adst_hbm.at[vsel2[p]], adb2[p], sem).wait()
        for g in range(CH // L):
            a = asb2[p][pl.ds(g * L, L)] + adb2[p][pl.ds(g * L, L)]
            a = jnp.where(a < 0.0, 0.2 * a, a)
            exb_v[pl.ds(g * L, L)] = jnp.exp(a)
        pltpu.sync_copy(exb_v, s_sp.at[vsel2[p]], add=True)

    fire_a(0, 0)

    def pair_a(k, _):
        i0 = 2 * k
        fire_a(i0 + 1, 1)
        compute_a(i0, 0)

        @pl.when(k < NCHA // 2 - 1)
        def _():
            fire_a(i0 + 2, 0)
        compute_a(i0 + 1, 1)
        return 0

    lax.fori_loop(0, NCHA // 2, pair_a, 0)
    plsc.subcore_barrier()

    @pl.when(sid == 0)
    def _():
        soff = pl.multiple_of(cid * NSEG, 8)
        pltpu.sync_copy(s_sp, sout_hbm.at[pl.ds(soff, NSEG)])
    plsc.subcore_barrier()

    for half in range(2):
        src_h = (src0_hbm, src1_hbm)[half]
        dst_h = (dst0_hbm, dst1_hbm)[half]
        @pl.when(sid < NS - 1)
        def _():
            off = pl.multiple_of(sid * 1248, 8)
            pltpu.sync_copy(z2_hbm.at[pl.ds(off, 1248)],
                            agg_sp.at[pl.ds(off, 1248)])

        @pl.when(sid == NS - 1)
        def _():
            pltpu.sync_copy(z2_hbm.at[pl.ds((NS - 1) * 1248, NSEG - (NS - 1) * 1248)],
                            agg_sp.at[pl.ds((NS - 1) * 1248, NSEG - (NS - 1) * 1248)])
        plsc.subcore_barrier()

        def fire_b(i, p):
            base = pl.multiple_of(cid * EPC + sid * EPT_B + i * CH, CH)
            pltpu.sync_copy(u_hbm.at[pl.ds(base, CH)], uc_v)
            pltpu.sync_copy(v_hbm.at[pl.ds(base, CH)], vc_v)
            pltpu.sync_copy(ea_hbm.at[pl.ds(base, CH)], eac2[p])
            for g in range(CH // L):
                ul = uc_v[pl.ds(g * L, L)]
                vl = vc_v[pl.ds(g * L, L)]
                su = jnp.where(ul >= NODE, ONE, ZERO)
                dv = jnp.where(vl >= NODE, ONE, ZERO)
                usel = ul + NTOT * dv
                vsel = vl + NTOT * su
                usel2[p][pl.ds(g * L, L)] = usel
                vsel2[p][pl.ds(g * L, L)] = vsel
                sgi2[p][pl.ds(g * L, L)] = cid * NSEG + vsel
                qoff2[p][pl.ds(g * L, L)] = (3 * su + 2 * dv - 4 * su * dv) * D + half * HD
            pltpu.async_copy(src_h.at[usel2[p]], srcrows2[p], sem)
            pltpu.async_copy(dst_h.at[vsel2[p]], dstrows2[p], sem)
            pltpu.async_copy(sout_hbm.at[sgi2[p]], sgb2[p], sem)

        def compute_b(i, p):
            pltpu.make_async_copy(src_h.at[usel2[p]], srcrows2[p], sem).wait()
            pltpu.make_async_copy(dst_h.at[vsel2[p]], dstrows2[p], sem).wait()
            pltpu.make_async_copy(sout_hbm.at[sgi2[p]], sgb2[p], sem).wait()
            srcrows_v = srcrows2[p]
            dstrows_v = dstrows2[p]
            for g in range(CH // L):
                le = iot + g * L
                a = (plsc.load_gather(srcrows_v, [le, jnp.full((L,), HD * 2, jnp.int32)])
                     + plsc.load_gather(dstrows_v, [le, jnp.full((L,), HD, jnp.int32)]))
                a = jnp.where(a < 0.0, 0.2 * a, a)
                sg = sgb2[p][pl.ds(g * L, L)]
                wbuf_v[pl.ds(g * L, L)] = jnp.exp(a) / (sg + 1e-16)

            def edge_grp(g, _):
                gb = g * L
                w16 = wbuf_v[pl.ds(gb, L)]
                ea16 = eac2[p][pl.ds(gb, L)]
                qo16 = qoff2[p][pl.ds(gb, L)]
                for ee in range(L):
                    e = gb + ee
                    we = w16[ee]
                    eae = ea16[ee]
                    qo = qo16[ee]
                    for j in range(HD // L):
                        hv = srcrows_v[e, pl.ds(j * L, L)]
                        p1 = srcrows_v[e, pl.ds(HD + j * L, L)]
                        p2 = dstrows_v[e, pl.ds(j * L, L)]
                        vtv = vt_v[pl.ds(qo + j * L, L)]
                        msg_v[e, pl.ds(j * L, L)] = hv * (p1 + p2 + eae * vtv) * we
                return 0

            lax.fori_loop(0, CH // L, edge_grp, 0)
            pltpu.sync_copy(msg_v, agg_sp.at[vsel2[p]], add=True)

        fire_b(0, 0)

        def pair_b(k, _):
            i0 = 2 * k
            fire_b(i0 + 1, 1)
            compute_b(i0, 0)
            fire_b(i0 + 2, 0)
            compute_b(i0 + 1, 1)
            return 0

        lax.fori_loop(0, (NCHB - 1) // 2, pair_b, 0)
        compute_b(NCHB - 1, 0)
        plsc.subcore_barrier()
        @pl.when(sid < NS - 1)
        def _():
            off = pl.multiple_of(sid * 1248, 8)
            pltpu.sync_copy(agg_sp.at[pl.ds(off, 1248)],
                            out_hbm.at[cid, half, pl.ds(off, 1248)])

        @pl.when(sid == NS - 1)
        def _():
            pltpu.sync_copy(agg_sp.at[pl.ds((NS - 1) * 1248, NSEG - (NS - 1) * 1248)],
                            out_hbm.at[cid, half, pl.ds((NS - 1) * 1248, NSEG - (NS - 1) * 1248)])
        plsc.subcore_barrier()

def _sc_edges(u, v, ea, src0, src1, dst0, dst1, asrc, adst, vtf, z1, z2):
    mesh = plsc.VectorSubcoreMesh(core_axis_name="c", subcore_axis_name="s",
                                  num_cores=NC, num_subcores=NS)
    fn = functools.partial(
        pl.kernel,
        compiler_params=pltpu.CompilerParams(use_tc_tiling_on_sc=False,
                                             needs_layout_passes=False),
        out_type=(jax.ShapeDtypeStruct((NC, 2, NSEG, HD), jnp.float32),
                  jax.ShapeDtypeStruct((NC * NSEG,), jnp.float32)),
        mesh=mesh,
        scratch_types=[
            pltpu.VMEM((4 * D,), jnp.float32),
            pltpu.VMEM((CH,), jnp.int32),
            pltpu.VMEM((CH,), jnp.int32),
            pltpu.VMEM((CH,), jnp.float32),
            pltpu.VMEM((CH,), jnp.float32),
            pltpu.VMEM((CH, HD), jnp.float32),
            pltpu.VMEM((CH,), jnp.float32),
            pltpu.VMEM((CH,), jnp.float32),
            pltpu.VMEM((CH,), jnp.int32),
            pltpu.VMEM((CH,), jnp.int32),
            pltpu.VMEM((CH,), jnp.int32),
            pltpu.VMEM((CH,), jnp.int32),
            pltpu.VMEM((CH,), jnp.int32),
            pltpu.VMEM((CH,), jnp.int32),
            pltpu.VMEM((CH,), jnp.int32),
            pltpu.VMEM((CH,), jnp.int32),
            pltpu.VMEM((CH,), jnp.float32),
            pltpu.VMEM((CH,), jnp.float32),
            pltpu.VMEM((CH,), jnp.float32),
            pltpu.VMEM((CH,), jnp.float32),
            pltpu.VMEM((CH,), jnp.float32),
            pltpu.VMEM((CH,), jnp.float32),
            pltpu.VMEM((CH, SRCW), jnp.float32),
            pltpu.VMEM((CH, SRCW), jnp.float32),
            pltpu.VMEM((CH, DSTW), jnp.float32),
            pltpu.VMEM((CH, DSTW), jnp.float32),
            pltpu.VMEM_SHARED((NSEG,), jnp.float32),
            pltpu.VMEM_SHARED((NSEG, HD), jnp.float32),
            pltpu.SemaphoreType.DMA,
        ],
    )(_sc_body)
    aggp, _ = fn(u, v, ea, src0, src1, dst0, dst1, asrc, adst, vtf, z1, z2)
    return aggp


def _c1_body(av0, av1, h_ref, w1, w2, bb, kw, kb, qq, o_ref, ps_ref):
    hb = h_ref[...]
    for vv in range(2):
        avr = (av0, av1)[vv]
        aggr = jnp.concatenate(
            [avr[0, 0] + avr[1, 0], avr[0, 1] + avr[1, 1]], axis=-1)
        upd = (jnp.dot(aggr, w1[0, vv], preferred_element_type=jnp.float32)
               + jnp.dot(hb, w2[0, vv], preferred_element_type=jnp.float32)
               + bb[0, vv])
        o = jnp.tanh(jnp.maximum(upd, 0.0))
        o_ref[0, vv] = o
        kl = jnp.tanh(jnp.dot(o, kw[0], preferred_element_type=jnp.float32) + kb[0])
        ps_ref[0, 0, vv] = jnp.sum(qq[0] * kl, axis=0)


def _c1(aggp, hall, w1, w2, bb, kw, kb, qq):
    def off0(nt, i):
        return (0, 0, jnp.where(nt == 0, 0, 15000 // BC) + i, 0)

    def off1(nt, i):
        return (0, 0, jnp.where(nt == 0, 10000 // BC, 5000 // BC) + i, 0)

    return pl.pallas_call(
        _c1_body,
        grid=(2, NBC),
        in_specs=[
            pl.BlockSpec((2, 2, BC, HD), off0),
            pl.BlockSpec((2, 2, BC, HD), off1),
            pl.BlockSpec((BC, D), lambda nt, i: (nt * NBC + i, 0)),
            pl.BlockSpec((1, 2, D, D), lambda nt, i: (nt, 0, 0, 0)),
            pl.BlockSpec((1, 2, D, D), lambda nt, i: (nt, 0, 0, 0)),
            pl.BlockSpec((1, 2, 1, D), lambda nt, i: (nt, 0, 0, 0)),
            pl.BlockSpec((1, D, D), lambda nt, i: (nt, 0, 0)),
            pl.BlockSpec((1, 1, D), lambda nt, i: (nt, 0, 0)),
            pl.BlockSpec((1, 1, D), lambda nt, i: (nt, 0, 0)),
        ],
        out_specs=(
            pl.BlockSpec((1, 2, BC, D), lambda nt, i: (nt, 0, i, 0)),
            pl.BlockSpec((1, 1, 2, D), lambda nt, i: (nt, i, 0, 0)),
        ),
        out_shape=(jax.ShapeDtypeStruct((2, 2, NODE, D), jnp.float32),
                   jax.ShapeDtypeStruct((2, NBC, 2, D), jnp.float32)),
    )(aggp, aggp, hall, w1, w2, bb, kw, kb, qq)


def _c2_body(o_ref, ps_ref, out_ref):
    p0 = jnp.sum(ps_ref[0, :, 0, :]) * (1.0 / NODE)
    p1 = jnp.sum(ps_ref[0, :, 1, :]) * (1.0 / NODE)
    m = jnp.maximum(p0, p1)
    e0 = jnp.exp(p0 - m)
    e1 = jnp.exp(p1 - m)
    a0 = e0 / (e0 + e1)
    a1 = e1 / (e0 + e1)
    out_ref[...] = a0 * o_ref[0, 0] + a1 * o_ref[0, 1]


def _c2(o, ps):
    return pl.pallas_call(
        _c2_body,
        grid=(2, NBC),
        in_specs=[
            pl.BlockSpec((1, 2, BC, D), lambda nt, i: (nt, 0, i, 0)),
            pl.BlockSpec((1, NBC, 2, D), lambda nt, i: (nt, 0, 0, 0)),
        ],
        out_specs=pl.BlockSpec((BC, D), lambda nt, i: (nt * NBC + i, 0)),
        out_shape=jax.ShapeDtypeStruct((NTOT, D), jnp.float32),
    )(o, ps)


def kernel(x, edge_index, edge_attr, params):
    xb = x[0]
    u = edge_index[0, 0]
    v = edge_index[0, 1]
    eaf = edge_attr[0, :, 0]
    p = params

    epw = jnp.stack([p["ep_W_" + n][0] for n in NAMES])
    epb = jnp.stack([p["ep_b_" + n] for n in NAMES])
    attb = jnp.stack([p["att_b_" + n] for n in NAMES])
    a3 = jnp.stack([p["att_W_" + n][2 * D:] for n in NAMES])
    vt, cc = _pre2(epw, epb, attb, a3)

    zpad = jnp.zeros((NPAD - NODE, D), jnp.float32)
    x2 = jnp.stack([jnp.concatenate([xb[:NODE], zpad]),
                    jnp.concatenate([xb[NODE:], zpad])])
    w2 = jnp.stack([p["proj_W_s"], p["proj_W_t"]])
    b2 = jnp.stack([p["proj_b_s"], p["proj_b_t"]])[:, None, :]

    def wcat_for(nt):
        n2s = nt + "2s"
        n2t = nt + "2t"
        fs = "s2" + nt
        ft = "t2" + nt
        scal = jnp.zeros((D, D), jnp.float32)
        scal = scal.at[:, 0].set(p["lin_src_" + n2s])
        scal = scal.at[:, 1].set(p["lin_src_" + n2t])
        scal = scal.at[:, 2].set(p["lin_dst_" + fs])
        scal = scal.at[:, 3].set(p["lin_dst_" + ft])
        return jnp.concatenate([
            p["att_W_" + n2s][:D], p["att_W_" + n2t][:D],
            p["att_W_" + fs][D:2 * D], p["att_W_" + ft][D:2 * D], scal], axis=1)

    wcat = jnp.stack([wcat_for("s"), wcat_for("t")])
    ni = {n: i for i, n in enumerate(NAMES)}

    def bcat_for(nt):
        z = jnp.zeros((D,), jnp.float32)
        return jnp.concatenate([z, z, cc[ni["s2" + nt]], cc[ni["t2" + nt]], z])

    bcat = jnp.stack([bcat_for("s"), bcat_for("t")])[:, None, :]
    hout, pout = _pre1(x2, w2, b2, wcat, bcat)
    hs, ht = hout[0, :NODE], hout[1, :NODE]
    ps_, pt_ = pout[0, :NODE], pout[1, :NODE]
    hall = jnp.concatenate([hs, ht])

    z15 = jnp.zeros((NODE, SRCW - 2 * HD - 1), jnp.float32)

    def src_tab(h):
        c = slice(h * HD, h * HD + HD)
        blks = []
        for dtv in range(2):
            a_col = 512 + dtv
            p1c = slice(dtv * D + h * HD, dtv * D + h * HD + HD)
            for hh, pp in ((hs, ps_), (ht, pt_)):
                blks.append(jnp.concatenate(
                    [hh[:, c], pp[:, p1c], pp[:, a_col][:, None], z15], axis=1))
        return jnp.concatenate(blks)

    def dst_tab(h):
        blks = []
        for stv in range(2):
            a_col = 514 + stv
            p2c = slice(2 * D + stv * D + h * HD, 2 * D + stv * D + h * HD + HD)
            for pp in (ps_, pt_):
                blks.append(jnp.concatenate(
                    [pp[:, p2c], pp[:, a_col][:, None], z15], axis=1))
        return jnp.concatenate(blks)

    src0, src1 = src_tab(0), src_tab(1)
    dst0, dst1 = dst_tab(0), dst_tab(1)
    asrc = jnp.concatenate([ps_[:, 512], pt_[:, 512], ps_[:, 513], pt_[:, 513]])
    adst = jnp.concatenate([ps_[:, 514], pt_[:, 514], ps_[:, 515], pt_[:, 515]])
    vtf = vt.reshape(-1)
    z1 = jnp.zeros((NSEG,), jnp.float32)
    z2 = jnp.zeros((NSEG, HD), jnp.float32)

    aggp = _sc_edges(u, v, eaf, src0, src1, dst0, dst1, asrc, adst, vtf, z1, z2)

    vorder = {"s": ("s2s", "t2s"), "t": ("t2t", "s2t")}
    w1 = jnp.stack([jnp.stack([p["agg_W_" + n][:D] for n in vorder[nt]]) for nt in "st"])
    w2c = jnp.stack([jnp.stack([p["agg_W_" + n][D:] for n in vorder[nt]]) for nt in "st"])
    bb = jnp.stack([jnp.stack([p["agg_b_" + n][None, :] for n in vorder[nt]]) for nt in "st"])
    kw = jnp.stack([p["k_W_s"], p["k_W_t"]])
    kb = jnp.stack([p["k_b_s"], p["k_b_t"]])[:, None, :]
    qq = jnp.stack([p["q_s"], p["q_t"]])

    o, psum = _c1(aggp, hall, w1, w2c, bb, kw, kb, qq)
    out = _c2(o, psum)
    return out[None]
